# baseline (speedup 1.0000x reference)
import functools

import jax
import jax.numpy as jnp
from jax import lax
from jax.experimental import pallas as pl
from jax.experimental.pallas import tpu as pltpu
from jax.experimental.pallas import tpu_sc as plsc

_VOCAB = 1000
_B, _L = 1024, 50
_B0 = 768
_NC, _NS = 2, 16
_NW = _NC * _NS
_NBLK = _VOCAB // 128
_TAIL = _VOCAB - 128 * _NBLK
_LP = 56


def _make_gather(nb):
    bpw = nb // _NW
    slab_n = bpw * _LP
    nq = 2 * bpw
    mesh = plsc.VectorSubcoreMesh(core_axis_name="c", subcore_axis_name="s")

    @functools.partial(
        pl.kernel,
        mesh=mesh,
        out_type=jax.ShapeDtypeStruct((nb, _L, _VOCAB), jnp.float32),
        scratch_types=[
            pltpu.VMEM((slab_n,), jnp.int32),
            pltpu.VMEM((_L, 4, 128), jnp.float32),
            pltpu.VMEM((_L, 4, 128), jnp.float32),
            pltpu.VMEM((_L, 4, 128), jnp.float32),
            pltpu.VMEM((_L, 4, 128), jnp.float32),
        ] + [pltpu.SemaphoreType.DMA] * 12,
    )
    def gather_kernel(idxp_hbm, tlo_hbm, thi_hbm, out_hbm, slab,
                      buf0, buf1, buf2, buf3,
                      g0, g1, g2, g3, w0, w1, w2, w3, t0, t1, t2, t3):
        wid = lax.axis_index("s") * _NC + lax.axis_index("c")
        b0 = wid * bpw
        pltpu.sync_copy(idxp_hbm.at[pl.ds(b0 * _LP, slab_n)], slab)

        sets = ((buf0, g0, w0, t0, tlo_hbm, 0),
                (buf1, g1, w1, t1, thi_hbm, 1),
                (buf2, g2, w2, t2, tlo_hbm, 0),
                (buf3, g3, w3, t3, thi_hbm, 1))

        def start_gather(q, buf, g, table):
            pltpu.async_copy(
                table.at[slab.at[pl.ds((q // 2) * _LP, _L)]], buf, g)

        def complete(q, buf, g, w, tw, table, half):
            b = b0 + q // 2
            pltpu.make_async_copy(
                table.at[slab.at[pl.ds(0, _L)]], buf, g).wait()
            nblk = 4 if half == 0 else _NBLK - 4
            for c in range(nblk):
                pltpu.async_copy(
                    buf.at[:, c, :],
                    out_hbm.at[b, :, pl.ds((4 * half + c) * 128, 128)], w)
            if half == 1:
                for r in range(_L):
                    pltpu.async_copy(
                        buf.at[r, 3, pl.ds(0, _TAIL)],
                        out_hbm.at[b, r, pl.ds(128 * _NBLK, _TAIL)], tw)

        def wait_outputs(buf, w, tw, half):
            nblk = 4 if half == 0 else _NBLK - 4
            for c in range(nblk):
                pltpu.make_async_copy(
                    buf.at[:, c, :],
                    out_hbm.at[b0, :, pl.ds(c * 128, 128)], w).wait()
            if half == 1:
                for r in range(_L):
                    pltpu.make_async_copy(
                        buf.at[r, 3, pl.ds(0, _TAIL)],
                        out_hbm.at[b0, r, pl.ds(128 * _NBLK, _TAIL)],
                        tw).wait()

        for j, (buf, g, w, tw, table, half) in enumerate(sets):
            start_gather(j, buf, g, table)

        def body(t, carry):
            q = 4 * t
            for j, (buf, g, w, tw, table, half) in enumerate(sets):
                complete(q + j, buf, g, w, tw, table, half)
            for j, (buf, g, w, tw, table, half) in enumerate(sets):
                wait_outputs(buf, w, tw, half)
                start_gather(q + 4 + j, buf, g, table)
            return carry

        lax.fori_loop(0, nq // 4 - 1, body, 0)

        for j, (buf, g, w, tw, table, half) in enumerate(sets):
            complete(nq - 4 + j, buf, g, w, tw, table, half)
        for buf, g, w, tw, table, half in sets:
            wait_outputs(buf, w, tw, half)

    return gather_kernel


_gather_big = _make_gather(_B0)
_gather_small = _make_gather(_B - _B0)


@jax.jit
def kernel(idx, logits):
    table_p = jnp.pad(logits, ((0, 0), (0, 24))).reshape(_VOCAB, 8, 128)
    tlo, thi = table_p[:, :4], table_p[:, 4:]
    idxp = jnp.pad(idx, ((0, 0), (0, _LP - _L)))
    big = _gather_big(idxp[:_B0].reshape(-1), tlo, thi)
    small = _gather_small(idxp[_B0:].reshape(-1), tlo, thi)
    out = jnp.pad(big, ((0, _B - _B0), (0, 0), (0, 0)))
    return lax.dynamic_update_slice(out, small, (_B0, 0, 0))


# speedup vs baseline: 1.2459x; 1.2459x over previous
import functools

import jax
import jax.numpy as jnp
from jax import lax
from jax.experimental import pallas as pl
from jax.experimental.pallas import tpu as pltpu
from jax.experimental.pallas import tpu_sc as plsc

_VOCAB = 1000
_B, _L = 1024, 50
_NC, _NS = 2, 16
_NW = _NC * _NS
_BPW = _B // _NW
_NBLK = _VOCAB // 128
_TAIL = _VOCAB - 128 * _NBLK
_LP = 56
_SLAB = _BPW * _LP
_NQ = 2 * _BPW


def _make_gather():
    mesh = plsc.VectorSubcoreMesh(core_axis_name="c", subcore_axis_name="s")

    @functools.partial(
        pl.kernel,
        mesh=mesh,
        out_type=jax.ShapeDtypeStruct((_B, _L, _VOCAB), jnp.float32),
        scratch_types=[
            pltpu.VMEM((_SLAB,), jnp.int32),
            pltpu.VMEM((_L, 4, 128), jnp.float32),
            pltpu.VMEM((_L, 4, 128), jnp.float32),
            pltpu.VMEM((_L, 4, 128), jnp.float32),
            pltpu.VMEM((_L, 4, 128), jnp.float32),
        ] + [pltpu.SemaphoreType.DMA] * 12,
    )
    def gather_kernel(idxp_hbm, tlo_hbm, thi_hbm, out_hbm, slab,
                      buf0, buf1, buf2, buf3,
                      g0, g1, g2, g3, w0, w1, w2, w3, t0, t1, t2, t3):
        wid = lax.axis_index("s") * _NC + lax.axis_index("c")
        b0 = wid * _BPW
        pltpu.sync_copy(idxp_hbm.at[pl.ds(b0 * _LP, _SLAB)], slab)

        sets = ((buf0, g0, w0, t0, tlo_hbm, 0),
                (buf1, g1, w1, t1, thi_hbm, 1),
                (buf2, g2, w2, t2, tlo_hbm, 0),
                (buf3, g3, w3, t3, thi_hbm, 1))

        def start_gather(q, buf, g, table):
            pltpu.async_copy(
                table.at[slab.at[pl.ds((q // 2) * _LP, _L)]], buf, g)

        def complete(q, buf, g, w, tw, table, half):
            b = b0 + q // 2
            pltpu.make_async_copy(
                table.at[slab.at[pl.ds(0, _L)]], buf, g).wait()
            nblk = 4 if half == 0 else _NBLK - 4
            for c in range(nblk):
                pltpu.async_copy(
                    buf.at[:, c, :],
                    out_hbm.at[b, :, pl.ds((4 * half + c) * 128, 128)], w)
            if half == 1:
                for r in range(_L):
                    pltpu.async_copy(
                        buf.at[r, 3, pl.ds(0, _TAIL)],
                        out_hbm.at[b, r, pl.ds(128 * _NBLK, _TAIL)], tw)

        def wait_outputs(buf, w, tw, half):
            nblk = 4 if half == 0 else _NBLK - 4
            for c in range(nblk):
                pltpu.make_async_copy(
                    buf.at[:, c, :],
                    out_hbm.at[b0, :, pl.ds(c * 128, 128)], w).wait()
            if half == 1:
                for r in range(_L):
                    pltpu.make_async_copy(
                        buf.at[r, 3, pl.ds(0, _TAIL)],
                        out_hbm.at[b0, r, pl.ds(128 * _NBLK, _TAIL)],
                        tw).wait()

        for j, (buf, g, w, tw, table, half) in enumerate(sets):
            start_gather(j, buf, g, table)

        def body(t, carry):
            q = 4 * t
            for j, (buf, g, w, tw, table, half) in enumerate(sets):
                complete(q + j, buf, g, w, tw, table, half)
            for j, (buf, g, w, tw, table, half) in enumerate(sets):
                wait_outputs(buf, w, tw, half)
                start_gather(q + 4 + j, buf, g, table)
            return carry

        lax.fori_loop(0, _NQ // 4 - 1, body, 0)

        for j, (buf, g, w, tw, table, half) in enumerate(sets):
            complete(_NQ - 4 + j, buf, g, w, tw, table, half)
        for buf, g, w, tw, table, half in sets:
            wait_outputs(buf, w, tw, half)

    return gather_kernel


_gather = _make_gather()


@jax.jit
def kernel(idx, logits):
    table_p = jnp.pad(logits, ((0, 0), (0, 24))).reshape(_VOCAB, 8, 128)
    idxp = jnp.pad(idx, ((0, 0), (0, _LP - _L))).reshape(-1)
    return _gather(idxp, table_p[:, :4], table_p[:, 4:])
